# baseline (speedup 1.0000x reference)
import functools

import jax
import jax.numpy as jnp
from jax import lax
from jax.experimental import pallas as pl
from jax.experimental.pallas import tpu as pltpu
from jax.experimental.pallas import tpu_sc as plsc

NUM_CORES = 2
NUM_SUBCORES = 16
LANES = 16
NW = NUM_CORES * NUM_SUBCORES

BATCH = 16384
EMBED_DIM = 64
TAB_ROWS = 500000
BW = BATCH // NW
CHUNK = 128
NCHUNK = BW // CHUNK
HALF = BW // 2
HCH = HALF // CHUNK
GROUPS = HALF // LANES


def _body(pos_hbm, neg_hbm, tab_hbm, out_hbm,
          idx_p_v, idx_n_v, row_p_v, row_n_v, hof_p_v, hof_n_v,
          st_p, st_n, out_v, sem):
    wid = lax.axis_index("s") * NUM_CORES + lax.axis_index("c")

    pltpu.sync_copy(pos_hbm.at[pl.ds(wid * NCHUNK, NCHUNK)], idx_p_v)
    pltpu.sync_copy(neg_hbm.at[pl.ds(wid * NCHUNK, NCHUNK)], idx_n_v)

    for j in range(NCHUNK):
        for k in range(CHUNK // LANES):
            s = pl.ds(k * LANES, LANES)
            f = pl.ds(j * CHUNK + k * LANES, LANES)
            vp = idx_p_v[j, s]
            vn = idx_n_v[j, s]
            row_p_v[j, s] = lax.shift_right_logical(vp, 1)
            row_n_v[j, s] = lax.shift_right_logical(vn, 1)
            hof_p_v[f] = lax.shift_left(jnp.bitwise_and(vp, 1), 6)
            hof_n_v[f] = lax.shift_left(jnp.bitwise_and(vn, 1), 6)

    lane = lax.iota(jnp.int32, LANES)

    for h in range(2):
        copies = []
        for j in range(HCH):
            copies.append(pltpu.async_copy(
                tab_hbm.at[row_p_v.at[h * HCH + j]],
                st_p.at[pl.ds(j * CHUNK, CHUNK)], sem))
            copies.append(pltpu.async_copy(
                tab_hbm.at[row_n_v.at[h * HCH + j]],
                st_n.at[pl.ds(j * CHUNK, CHUNK)], sem))
        for c in copies:
            c.wait()

        def group(g, carry):
            r = g * LANES
            rvec = r + lane
            hp = hof_p_v[pl.ds(h * HALF + r, LANES)]
            hn = hof_n_v[pl.ds(h * HALF + r, LANES)]
            acc = (plsc.load_gather(st_p, [rvec, hp])
                   * plsc.load_gather(st_n, [rvec, hn]))

            def dstep(d, a):
                ap = plsc.load_gather(st_p, [rvec, hp + d])
                an = plsc.load_gather(st_n, [rvec, hn + d])
                return a + ap * an

            acc = lax.fori_loop(1, EMBED_DIM, dstep, acc)
            out_v[pl.ds(h * HALF + r, LANES)] = acc
            return carry

        lax.fori_loop(0, GROUPS, group, 0)

    pltpu.sync_copy(out_v, out_hbm.at[pl.ds(wid * BW, BW)])


_sc_call = functools.partial(
    pl.kernel,
    mesh=plsc.VectorSubcoreMesh(core_axis_name="c", subcore_axis_name="s"),
    out_type=jax.ShapeDtypeStruct((BATCH,), jnp.float32),
    compiler_params=pltpu.CompilerParams(
        use_tc_tiling_on_sc=True, needs_layout_passes=False),
    scratch_types=[
        pltpu.VMEM((NCHUNK, CHUNK), jnp.int32),
        pltpu.VMEM((NCHUNK, CHUNK), jnp.int32),
        pltpu.VMEM((NCHUNK, CHUNK), jnp.int32),
        pltpu.VMEM((NCHUNK, CHUNK), jnp.int32),
        pltpu.VMEM((BW,), jnp.int32),
        pltpu.VMEM((BW,), jnp.int32),
        pltpu.VMEM((HALF, CHUNK), jnp.float32),
        pltpu.VMEM((HALF, CHUNK), jnp.float32),
        pltpu.VMEM((BW,), jnp.float32),
        pltpu.SemaphoreType.DMA,
    ],
)(_body)


@jax.jit
def kernel(stock_pos, stock_neg, embeddings):
    pos = stock_pos.astype(jnp.int32).reshape(NW * NCHUNK, CHUNK)
    neg = stock_neg.astype(jnp.int32).reshape(NW * NCHUNK, CHUNK)
    tab = embeddings.reshape(TAB_ROWS, 2 * EMBED_DIM)
    return _sc_call(pos, neg, tab)


# speedup vs baseline: 1.3316x; 1.3316x over previous
import functools

import jax
import jax.numpy as jnp
from jax import lax
from jax.experimental import pallas as pl
from jax.experimental.pallas import tpu as pltpu
from jax.experimental.pallas import tpu_sc as plsc

NUM_CORES = 2
NUM_SUBCORES = 16
LANES = 16
NW = NUM_CORES * NUM_SUBCORES

BATCH = 16384
EMBED_DIM = 64
NUM_PROD = 1000000
ROWPAD = 128

BW = BATCH // NW
CHUNK = 128
NCHUNK = BW // CHUNK
HALF = BW // 2
HCH = HALF // CHUNK
GROUPS = HALF // LANES

TP = 2048


def _transpose_body(x_ref, y_ref):
    y_ref[:, 0:EMBED_DIM] = x_ref[...].T


_tc_transpose = pl.pallas_call(
    _transpose_body,
    grid=(pl.cdiv(NUM_PROD, TP),),
    in_specs=[pl.BlockSpec((EMBED_DIM, TP), lambda k: (0, k))],
    out_specs=pl.BlockSpec((TP, ROWPAD), lambda k: (k, 0)),
    out_shape=jax.ShapeDtypeStruct((NUM_PROD, ROWPAD), jnp.float32),
)


def _sc_body(pos_hbm, neg_hbm, tab_hbm, out_hbm,
             idx_p_v, idx_n_v, st_p, st_n, trans_v, out_v, sem):
    wid = lax.axis_index("s") * NUM_CORES + lax.axis_index("c")

    pltpu.sync_copy(pos_hbm.at[pl.ds(wid * NCHUNK, NCHUNK)], idx_p_v)
    pltpu.sync_copy(neg_hbm.at[pl.ds(wid * NCHUNK, NCHUNK)], idx_n_v)

    lane = lax.iota(jnp.int32, LANES)

    for h in range(2):
        copies = []
        for j in range(HCH):
            copies.append(pltpu.async_copy(
                tab_hbm.at[idx_p_v.at[h * HCH + j]],
                st_p.at[pl.ds(j * CHUNK, CHUNK)], sem))
            copies.append(pltpu.async_copy(
                tab_hbm.at[idx_n_v.at[h * HCH + j]],
                st_n.at[pl.ds(j * CHUNK, CHUNK)], sem))
        for c in copies:
            c.wait()

        def group(g, carry):
            r = g * LANES
            for i in range(LANES):
                b = r + i
                p = st_p[b, pl.ds(0, LANES)] * st_n[b, pl.ds(0, LANES)]
                for j in range(1, EMBED_DIM // LANES):
                    p = p + (st_p[b, pl.ds(j * LANES, LANES)]
                             * st_n[b, pl.ds(j * LANES, LANES)])
                plsc.store_scatter(trans_v, [lane * LANES + i], p)
            acc = trans_v[pl.ds(0, LANES)]
            for l in range(1, LANES):
                acc = acc + trans_v[pl.ds(l * LANES, LANES)]
            out_v[pl.ds(h * HALF + r, LANES)] = acc
            return carry

        lax.fori_loop(0, GROUPS, group, 0)

    pltpu.sync_copy(out_v, out_hbm.at[pl.ds(wid * BW, BW)])


_sc_call = functools.partial(
    pl.kernel,
    mesh=plsc.VectorSubcoreMesh(core_axis_name="c", subcore_axis_name="s"),
    out_type=jax.ShapeDtypeStruct((BATCH,), jnp.float32),
    compiler_params=pltpu.CompilerParams(
        use_tc_tiling_on_sc=True, needs_layout_passes=False),
    scratch_types=[
        pltpu.VMEM((NCHUNK, CHUNK), jnp.int32),
        pltpu.VMEM((NCHUNK, CHUNK), jnp.int32),
        pltpu.VMEM((HALF, ROWPAD), jnp.float32),
        pltpu.VMEM((HALF, ROWPAD), jnp.float32),
        pltpu.VMEM((LANES * LANES,), jnp.float32),
        pltpu.VMEM((BW,), jnp.float32),
        pltpu.SemaphoreType.DMA,
    ],
)(_sc_body)


@jax.jit
def kernel(stock_pos, stock_neg, embeddings):
    pos = stock_pos.astype(jnp.int32).reshape(NW * NCHUNK, CHUNK)
    neg = stock_neg.astype(jnp.int32).reshape(NW * NCHUNK, CHUNK)
    tab = _tc_transpose(embeddings.T)
    return _sc_call(pos, neg, tab)


# speedup vs baseline: 2.2483x; 1.6884x over previous
import functools

import jax
import jax.numpy as jnp
from jax import lax
from jax.experimental import pallas as pl
from jax.experimental.pallas import tpu as pltpu
from jax.experimental.pallas import tpu_sc as plsc

NUM_CORES = 2
NUM_SUBCORES = 16
LANES = 16
NW = NUM_CORES * NUM_SUBCORES

BATCH = 16384
EMBED_DIM = 64
NUM_PROD = 1000000
ROWPAD = 128

BW = BATCH // NW
CHUNK = 128
NCHUNK = BW // CHUNK
HALF = BW // 2
HCH = HALF // CHUNK
GROUPS = HALF // LANES

TP = 8192
TPO = TP // 2
NROWS = pl.cdiv(NUM_PROD, TP) * TPO


def _transpose_body(x_ref, y_ref):
    y_ref[:, 0:EMBED_DIM] = x_ref[:, 0:TPO].T
    y_ref[:, EMBED_DIM:ROWPAD] = x_ref[:, TPO:TP].T


_tc_transpose = pl.pallas_call(
    _transpose_body,
    grid=(pl.cdiv(NUM_PROD, TP),),
    in_specs=[pl.BlockSpec((EMBED_DIM, TP), lambda k: (0, k))],
    out_specs=pl.BlockSpec((TPO, ROWPAD), lambda k: (k, 0)),
    out_shape=jax.ShapeDtypeStruct((NROWS, ROWPAD), jnp.float32),
    compiler_params=pltpu.CompilerParams(
        dimension_semantics=("parallel",)),
)


def _sc_body(pos_hbm, neg_hbm, tab_hbm, out_hbm,
             idx_p_v, idx_n_v, row_p_v, row_n_v, sel_p_v, sel_n_v,
             st_p, st_n, trans_v, out_v, sem):
    wid = lax.axis_index("s") * NUM_CORES + lax.axis_index("c")

    pltpu.sync_copy(pos_hbm.at[pl.ds(wid * NCHUNK, NCHUNK)], idx_p_v)
    pltpu.sync_copy(neg_hbm.at[pl.ds(wid * NCHUNK, NCHUNK)], idx_n_v)

    for c in range(NCHUNK):
        for s in range(CHUNK // LANES):
            for iv, rv, sv in ((idx_p_v, row_p_v, sel_p_v),
                               (idx_n_v, row_n_v, sel_n_v)):
                v = iv[c, pl.ds(s * LANES, LANES)]
                row = ((v >> 13) << 12) + (v & 4095)
                half = (v >> 12) & 1
                rv[c, pl.ds(s * LANES, LANES)] = row
                sv[pl.ds(c * CHUNK + s * LANES, LANES)] = (
                    half.astype(jnp.float32))

    lane = lax.iota(jnp.int32, LANES)

    for h in range(2):
        copies = []
        for j in range(HCH):
            copies.append(pltpu.async_copy(
                tab_hbm.at[row_p_v.at[h * HCH + j]],
                st_p.at[pl.ds(j * CHUNK, CHUNK)], sem))
            copies.append(pltpu.async_copy(
                tab_hbm.at[row_n_v.at[h * HCH + j]],
                st_n.at[pl.ds(j * CHUNK, CHUNK)], sem))
        for c in copies:
            c.wait()

        def group(g, carry):
            r = g * LANES
            mpg = sel_p_v[pl.ds(h * HALF + r, LANES)]
            mng = sel_n_v[pl.ds(h * HALF + r, LANES)]
            for i in range(LANES):
                b = r + i
                mp = mpg[i]
                mn = mng[i]
                acc = None
                for j in range(EMBED_DIM // LANES):
                    plo = st_p[b, pl.ds(j * LANES, LANES)]
                    phi = st_p[b, pl.ds(EMBED_DIM + j * LANES, LANES)]
                    nlo = st_n[b, pl.ds(j * LANES, LANES)]
                    nhi = st_n[b, pl.ds(EMBED_DIM + j * LANES, LANES)]
                    pv = plo + (phi - plo) * mp
                    nv = nlo + (nhi - nlo) * mn
                    acc = pv * nv if acc is None else acc + pv * nv
                plsc.store_scatter(trans_v, [lane * LANES + i], acc)
            accv = trans_v[pl.ds(0, LANES)]
            for l in range(1, LANES):
                accv = accv + trans_v[pl.ds(l * LANES, LANES)]
            out_v[pl.ds(h * HALF + r, LANES)] = accv
            return carry

        lax.fori_loop(0, GROUPS, group, 0)

    pltpu.sync_copy(out_v, out_hbm.at[pl.ds(wid * BW, BW)])


_sc_call = functools.partial(
    pl.kernel,
    mesh=plsc.VectorSubcoreMesh(core_axis_name="c", subcore_axis_name="s"),
    out_type=jax.ShapeDtypeStruct((BATCH,), jnp.float32),
    compiler_params=pltpu.CompilerParams(
        use_tc_tiling_on_sc=True, needs_layout_passes=False),
    scratch_types=[
        pltpu.VMEM((NCHUNK, CHUNK), jnp.int32),
        pltpu.VMEM((NCHUNK, CHUNK), jnp.int32),
        pltpu.VMEM((NCHUNK, CHUNK), jnp.int32),
        pltpu.VMEM((NCHUNK, CHUNK), jnp.int32),
        pltpu.VMEM((BW,), jnp.float32),
        pltpu.VMEM((BW,), jnp.float32),
        pltpu.VMEM((HALF, ROWPAD), jnp.float32),
        pltpu.VMEM((HALF, ROWPAD), jnp.float32),
        pltpu.VMEM((LANES * LANES,), jnp.float32),
        pltpu.VMEM((BW,), jnp.float32),
        pltpu.SemaphoreType.DMA,
    ],
)(_sc_body)


@jax.jit
def kernel(stock_pos, stock_neg, embeddings):
    pos = stock_pos.astype(jnp.int32).reshape(NW * NCHUNK, CHUNK)
    neg = stock_neg.astype(jnp.int32).reshape(NW * NCHUNK, CHUNK)
    tab = _tc_transpose(embeddings.T)
    return _sc_call(pos, neg, tab)


# speedup vs baseline: 2.5411x; 1.1302x over previous
import functools

import jax
import jax.numpy as jnp
from jax import lax
from jax.experimental import pallas as pl
from jax.experimental.pallas import tpu as pltpu
from jax.experimental.pallas import tpu_sc as plsc

NUM_CORES = 2
NUM_SUBCORES = 16
LANES = 16
NW = NUM_CORES * NUM_SUBCORES

BATCH = 16384
EMBED_DIM = 64
NUM_PROD = 1000000
ROWPAD = 128

BW = BATCH // NW
CHUNK = 128
NCHUNK = BW // CHUNK
HALF = BW // 2
HCH = HALF // CHUNK
GROUPS = HALF // LANES

TP = 16384
TPO = TP // 2
NROWS = pl.cdiv(NUM_PROD, TP) * TPO
LOG_TP = TP.bit_length() - 1
LOG_TPO = LOG_TP - 1


def _transpose_body(x_ref, y_ref):
    y_ref[...] = jnp.concatenate(
        [x_ref[:, 0:TPO].T, x_ref[:, TPO:TP].T], axis=1)


_tc_transpose = pl.pallas_call(
    _transpose_body,
    grid=(pl.cdiv(NUM_PROD, TP),),
    in_specs=[pl.BlockSpec((EMBED_DIM, TP), lambda k: (0, k))],
    out_specs=pl.BlockSpec((TPO, ROWPAD), lambda k: (k, 0)),
    out_shape=jax.ShapeDtypeStruct((NROWS, ROWPAD), jnp.float32),
    compiler_params=pltpu.CompilerParams(
        dimension_semantics=("parallel",)),
)


def _sc_body(pos_hbm, neg_hbm, tab_hbm, out_hbm,
             idx_p_v, idx_n_v, row_p_v, row_n_v, sel_p_v, sel_n_v,
             st_p, st_n, trans_v, out_v, sem):
    wid = lax.axis_index("s") * NUM_CORES + lax.axis_index("c")

    pltpu.sync_copy(pos_hbm.at[pl.ds(wid * NCHUNK, NCHUNK)], idx_p_v)
    pltpu.sync_copy(neg_hbm.at[pl.ds(wid * NCHUNK, NCHUNK)], idx_n_v)

    for c in range(NCHUNK):
        for s in range(CHUNK // LANES):
            for iv, rv, sv in ((idx_p_v, row_p_v, sel_p_v),
                               (idx_n_v, row_n_v, sel_n_v)):
                v = iv[c, pl.ds(s * LANES, LANES)]
                row = ((v >> LOG_TP) << LOG_TPO) + (v & (TPO - 1))
                half = (v >> LOG_TPO) & 1
                rv[c, pl.ds(s * LANES, LANES)] = row
                sv[pl.ds(c * CHUNK + s * LANES, LANES)] = (
                    half.astype(jnp.float32))

    lane = lax.iota(jnp.int32, LANES)

    for h in range(2):
        copies = []
        for j in range(HCH):
            copies.append(pltpu.async_copy(
                tab_hbm.at[row_p_v.at[h * HCH + j]],
                st_p.at[pl.ds(j * CHUNK, CHUNK)], sem))
            copies.append(pltpu.async_copy(
                tab_hbm.at[row_n_v.at[h * HCH + j]],
                st_n.at[pl.ds(j * CHUNK, CHUNK)], sem))
        for c in copies:
            c.wait()

        def group(g, carry):
            r = g * LANES
            mpg = sel_p_v[pl.ds(h * HALF + r, LANES)]
            mng = sel_n_v[pl.ds(h * HALF + r, LANES)]
            for i in range(LANES):
                b = r + i
                mp = mpg[i]
                mn = mng[i]
                acc = None
                for j in range(EMBED_DIM // LANES):
                    plo = st_p[b, pl.ds(j * LANES, LANES)]
                    phi = st_p[b, pl.ds(EMBED_DIM + j * LANES, LANES)]
                    nlo = st_n[b, pl.ds(j * LANES, LANES)]
                    nhi = st_n[b, pl.ds(EMBED_DIM + j * LANES, LANES)]
                    pv = plo + (phi - plo) * mp
                    nv = nlo + (nhi - nlo) * mn
                    acc = pv * nv if acc is None else acc + pv * nv
                plsc.store_scatter(trans_v, [lane * LANES + i], acc)
            accv = trans_v[pl.ds(0, LANES)]
            for l in range(1, LANES):
                accv = accv + trans_v[pl.ds(l * LANES, LANES)]
            out_v[pl.ds(h * HALF + r, LANES)] = accv
            return carry

        lax.fori_loop(0, GROUPS, group, 0)

    pltpu.sync_copy(out_v, out_hbm.at[pl.ds(wid * BW, BW)])


_sc_call = functools.partial(
    pl.kernel,
    mesh=plsc.VectorSubcoreMesh(core_axis_name="c", subcore_axis_name="s"),
    out_type=jax.ShapeDtypeStruct((BATCH,), jnp.float32),
    compiler_params=pltpu.CompilerParams(
        use_tc_tiling_on_sc=True, needs_layout_passes=False),
    scratch_types=[
        pltpu.VMEM((NCHUNK, CHUNK), jnp.int32),
        pltpu.VMEM((NCHUNK, CHUNK), jnp.int32),
        pltpu.VMEM((NCHUNK, CHUNK), jnp.int32),
        pltpu.VMEM((NCHUNK, CHUNK), jnp.int32),
        pltpu.VMEM((BW,), jnp.float32),
        pltpu.VMEM((BW,), jnp.float32),
        pltpu.VMEM((HALF, ROWPAD), jnp.float32),
        pltpu.VMEM((HALF, ROWPAD), jnp.float32),
        pltpu.VMEM((LANES * LANES,), jnp.float32),
        pltpu.VMEM((BW,), jnp.float32),
        pltpu.SemaphoreType.DMA,
    ],
)(_sc_body)


@jax.jit
def kernel(stock_pos, stock_neg, embeddings):
    pos = stock_pos.astype(jnp.int32).reshape(NW * NCHUNK, CHUNK)
    neg = stock_neg.astype(jnp.int32).reshape(NW * NCHUNK, CHUNK)
    tab = _tc_transpose(embeddings.T)
    return _sc_call(pos, neg, tab)


# speedup vs baseline: 2.6815x; 1.0553x over previous
import functools

import jax
import jax.numpy as jnp
from jax import lax
from jax.experimental import pallas as pl
from jax.experimental.pallas import tpu as pltpu
from jax.experimental.pallas import tpu_sc as plsc

NUM_CORES = 2
NUM_SUBCORES = 16
LANES = 16
NW = NUM_CORES * NUM_SUBCORES

BATCH = 16384
EMBED_DIM = 64
NUM_PROD = 1000000
ROWPAD = 128

BW = BATCH // NW
CHUNK = 128
NCHUNK = BW // CHUNK
HALF = BW // 2
HCH = HALF // CHUNK
GROUPS = HALF // LANES

TP = 32768
TPO = TP // 2
NROWS = pl.cdiv(NUM_PROD, TP) * TPO
LOG_TP = TP.bit_length() - 1
LOG_TPO = LOG_TP - 1


def _transpose_body(x_ref, y_ref):
    y_ref[...] = jnp.concatenate(
        [x_ref[:, 0:TPO].T, x_ref[:, TPO:TP].T], axis=1)


_tc_transpose = pl.pallas_call(
    _transpose_body,
    grid=(pl.cdiv(NUM_PROD, TP),),
    in_specs=[pl.BlockSpec((EMBED_DIM, TP), lambda k: (0, k))],
    out_specs=pl.BlockSpec((TPO, ROWPAD), lambda k: (k, 0)),
    out_shape=jax.ShapeDtypeStruct((NROWS, ROWPAD), jnp.float32),
    compiler_params=pltpu.CompilerParams(
        dimension_semantics=("parallel",)),
)


def _sc_body(pos_hbm, neg_hbm, tab_hbm, out_hbm,
             idx_p_v, idx_n_v, row_p_v, row_n_v, sel_p_v, sel_n_v,
             st_p, st_n, trans_v, out_v, sem):
    wid = lax.axis_index("s") * NUM_CORES + lax.axis_index("c")

    pltpu.sync_copy(pos_hbm.at[pl.ds(wid * NCHUNK, NCHUNK)], idx_p_v)
    pltpu.sync_copy(neg_hbm.at[pl.ds(wid * NCHUNK, NCHUNK)], idx_n_v)

    for c in range(NCHUNK):
        for s in range(CHUNK // LANES):
            for iv, rv, sv in ((idx_p_v, row_p_v, sel_p_v),
                               (idx_n_v, row_n_v, sel_n_v)):
                v = iv[c, pl.ds(s * LANES, LANES)]
                row = ((v >> LOG_TP) << LOG_TPO) + (v & (TPO - 1))
                half = (v >> LOG_TPO) & 1
                rv[c, pl.ds(s * LANES, LANES)] = row
                sv[pl.ds(c * CHUNK + s * LANES, LANES)] = (
                    half.astype(jnp.float32))

    lane = lax.iota(jnp.int32, LANES)

    for h in range(2):
        copies = []
        for j in range(HCH):
            copies.append(pltpu.async_copy(
                tab_hbm.at[row_p_v.at[h * HCH + j]],
                st_p.at[pl.ds(j * CHUNK, CHUNK)], sem))
            copies.append(pltpu.async_copy(
                tab_hbm.at[row_n_v.at[h * HCH + j]],
                st_n.at[pl.ds(j * CHUNK, CHUNK)], sem))
        for c in copies:
            c.wait()

        def group(g, carry):
            r = g * LANES
            mpg = sel_p_v[pl.ds(h * HALF + r, LANES)]
            mng = sel_n_v[pl.ds(h * HALF + r, LANES)]
            for i in range(LANES):
                b = r + i
                mp = mpg[i]
                mn = mng[i]
                acc = None
                for j in range(EMBED_DIM // LANES):
                    plo = st_p[b, pl.ds(j * LANES, LANES)]
                    phi = st_p[b, pl.ds(EMBED_DIM + j * LANES, LANES)]
                    nlo = st_n[b, pl.ds(j * LANES, LANES)]
                    nhi = st_n[b, pl.ds(EMBED_DIM + j * LANES, LANES)]
                    pv = plo + (phi - plo) * mp
                    nv = nlo + (nhi - nlo) * mn
                    acc = pv * nv if acc is None else acc + pv * nv
                plsc.store_scatter(trans_v, [lane * LANES + i], acc)
            accv = trans_v[pl.ds(0, LANES)]
            for l in range(1, LANES):
                accv = accv + trans_v[pl.ds(l * LANES, LANES)]
            out_v[pl.ds(h * HALF + r, LANES)] = accv
            return carry

        lax.fori_loop(0, GROUPS, group, 0)

    pltpu.sync_copy(out_v, out_hbm.at[pl.ds(wid * BW, BW)])


_sc_call = functools.partial(
    pl.kernel,
    mesh=plsc.VectorSubcoreMesh(core_axis_name="c", subcore_axis_name="s"),
    out_type=jax.ShapeDtypeStruct((BATCH,), jnp.float32),
    compiler_params=pltpu.CompilerParams(
        use_tc_tiling_on_sc=True, needs_layout_passes=False),
    scratch_types=[
        pltpu.VMEM((NCHUNK, CHUNK), jnp.int32),
        pltpu.VMEM((NCHUNK, CHUNK), jnp.int32),
        pltpu.VMEM((NCHUNK, CHUNK), jnp.int32),
        pltpu.VMEM((NCHUNK, CHUNK), jnp.int32),
        pltpu.VMEM((BW,), jnp.float32),
        pltpu.VMEM((BW,), jnp.float32),
        pltpu.VMEM((HALF, ROWPAD), jnp.float32),
        pltpu.VMEM((HALF, ROWPAD), jnp.float32),
        pltpu.VMEM((LANES * LANES,), jnp.float32),
        pltpu.VMEM((BW,), jnp.float32),
        pltpu.SemaphoreType.DMA,
    ],
)(_sc_body)


@jax.jit
def kernel(stock_pos, stock_neg, embeddings):
    pos = stock_pos.astype(jnp.int32).reshape(NW * NCHUNK, CHUNK)
    neg = stock_neg.astype(jnp.int32).reshape(NW * NCHUNK, CHUNK)
    tab = _tc_transpose(embeddings.T)
    return _sc_call(pos, neg, tab)
